# Initial kernel scaffold; baseline (speedup 1.0000x reference)
import jax
import jax.numpy as jnp
from jax.experimental import pallas as pl


def kernel(x, edge_index):
    raise NotImplementedError("write your pallas kernel here")


# speedup vs baseline: 2.4571x; 2.4571x over previous
import functools

import jax
import jax.numpy as jnp
from jax import lax
from jax.experimental import pallas as pl
from jax.experimental.pallas import tpu as pltpu
from jax.experimental.pallas import tpu_sc as plsc

_GATHER_DNUMS = lax.GatherDimensionNumbers(
    offset_dims=(), collapsed_slice_dims=(0,), start_index_map=(0,))


def _shuffle(v, idx):
    return lax.gather(v, idx[:, None], _GATHER_DNUMS, slice_sizes=(1,),
                      mode=lax.GatherScatterMode.PROMISE_IN_BOUNDS)


D = 128
L = 16
NC, NS = 2, 16
NW = NC * NS
C = 80


def _make_score_kernel(E):
    EW = E // NW
    nchunk = EW // C

    mesh = plsc.VectorSubcoreMesh(core_axis_name="c", subcore_axis_name="s")

    @functools.partial(
        pl.kernel,
        mesh=mesh,
        out_type=jax.ShapeDtypeStruct((E,), jnp.float32),
        scratch_types=[
            pltpu.VMEM((C,), jnp.int32),
            pltpu.VMEM((C,), jnp.int32),
            pltpu.VMEM((C, D), jnp.float32),
            pltpu.VMEM((C, D), jnp.float32),
            pltpu.VMEM((C,), jnp.float32),
            pltpu.SemaphoreType.DMA,
        ],
    )
    def score_k(x_hbm, src_hbm, dst_hbm, out_hbm,
                sidx, didx, srows, drows, outv, sem):
        wid = lax.axis_index("s") * NC + lax.axis_index("c")
        base0 = wid * EW

        def chunk_body(g, carry):
            base = pl.multiple_of(base0 + g * C, 8)
            pltpu.sync_copy(src_hbm.at[pl.ds(base, C)], sidx)
            pltpu.sync_copy(dst_hbm.at[pl.ds(base, C)], didx)
            pltpu.async_copy(x_hbm.at[sidx], srows, sem).wait()
            pltpu.async_copy(x_hbm.at[didx], drows, sem).wait()

            lanes = lax.iota(jnp.int32, L)

            def grp_body(jj, carry2):
                vec = jnp.zeros((L,), jnp.float32)
                for l in range(L):
                    j = jj * L + l
                    acc = jnp.zeros((L,), jnp.float32)
                    for k in range(D // L):
                        a = srows[j, pl.ds(k * L, L)]
                        b = drows[j, pl.ds(k * L, L)]
                        acc = acc + a * b
                    for s in (8, 4, 2, 1):
                        acc = acc + _shuffle(acc, lanes ^ s)
                    vec = jnp.where(lanes == l, acc, vec)
                outv[pl.ds(jj * L, L)] = vec
                return carry2

            lax.fori_loop(0, C // L, grp_body, 0, unroll=False)
            pltpu.sync_copy(outv, out_hbm.at[pl.ds(base, C)])
            return carry

        lax.fori_loop(0, nchunk, chunk_body, 0, unroll=False)

    return score_k


def kernel(x, edge_index):
    E = edge_index.shape[1]
    ei = edge_index.astype(jnp.int32)
    src = ei[0]
    dst = ei[1]

    step = NW * C
    Ep = ((E + step - 1) // step) * step
    if Ep != E:
        src = jnp.pad(src, (0, Ep - E))
        dst = jnp.pad(dst, (0, Ep - E))

    score = _make_score_kernel(Ep)(x, src, dst)
    return score[:E].reshape(E, 1)
